# baseline (speedup 1.0000x reference)
import functools

import jax
import jax.numpy as jnp
from jax import lax
from jax.experimental import pallas as pl
from jax.experimental.pallas import tpu as pltpu
from jax.experimental.pallas import tpu_sc as plsc

EMBED = 64
SCALE = 8.0

_info = plsc.get_sparse_core_info()
_NC, _NS, _L = _info.num_cores, _info.num_subcores, _info.num_lanes
_NW = _NC * _NS


@functools.partial(jax.jit, static_argnames=("b_per_w", "chunk"))
def _lookup(x_flat, table, b_per_w, chunk):
    n_chunks = b_per_w // chunk
    mesh = plsc.VectorSubcoreMesh(core_axis_name="c", subcore_axis_name="s")

    @functools.partial(
        pl.kernel,
        out_type=jax.ShapeDtypeStruct((x_flat.shape[0], 2 * EMBED), jnp.float32),
        mesh=mesh,
        scratch_types=[
            pltpu.VMEM((chunk,), jnp.int32),
            pltpu.VMEM((chunk,), jnp.int32),
            pltpu.VMEM((chunk, EMBED), jnp.float32),
            pltpu.VMEM((chunk, EMBED), jnp.float32),
            pltpu.VMEM((chunk, 2 * EMBED), jnp.float32),
            pltpu.VMEM((chunk, 2 * EMBED), jnp.float32),
            pltpu.SemaphoreType.DMA,
            pltpu.SemaphoreType.DMA,
        ],
        compiler_params=pltpu.CompilerParams(use_tc_tiling_on_sc=False),
    )
    def k(x_hbm, table_hbm, out_hbm, idx0, idx1, rows0, rows1, o0, o1,
          sem0, sem1):
        wid = lax.axis_index("s") * _NC + lax.axis_index("c")
        base = wid * b_per_w
        idx_v = (idx0, idx1)
        rows_v = (rows0, rows1)
        out_v = (o0, o1)
        sems = (sem0, sem1)
        zeros = jnp.zeros((_L,), jnp.float32)

        def zero_pad(r, c2):
            for b in range(2):
                for c in range(EMBED // _L):
                    out_v[b][r, pl.ds(EMBED + c * _L, _L)] = zeros
            return c2

        lax.fori_loop(0, chunk, zero_pad, 0)

        def start_gather(g, b):
            off = base + g * chunk
            pltpu.sync_copy(x_hbm.at[pl.ds(off, chunk)], idx_v[b])
            return pltpu.async_copy(table_hbm.at[idx_v[b]], rows_v[b], sems[b])

        def scale_and_store(g, b):
            rv = rows_v[b]
            ov = out_v[b]

            def scale_rows(r2, c2):
                for u in range(4):
                    for c in range(EMBED // _L):
                        sl = pl.ds(c * _L, _L)
                        ov[4 * r2 + u, sl] = rv[4 * r2 + u, sl] * SCALE
                return c2

            lax.fori_loop(0, chunk // 4, scale_rows, 0)
            pltpu.sync_copy(ov, out_hbm.at[pl.ds(base + g * chunk, chunk)])

        start_gather(0, 0)

        def pair_body(t, carry):
            for b in range(2):
                g = 2 * t + b
                pltpu.make_async_copy(
                    table_hbm.at[idx_v[b]], rows_v[b], sems[b]
                ).wait()

                @pl.when(g + 1 < n_chunks)
                def _prefetch():
                    start_gather(g + 1, 1 - b)

                scale_and_store(g, b)
            return carry

        lax.fori_loop(0, n_chunks // 2, pair_body, 0)

    return k(x_flat, table)


def kernel(x, embedding_table):
    orig_shape = x.shape
    x_flat = x.reshape(-1).astype(jnp.int32)
    b = x_flat.shape[0]
    b_per_w = b // _NW
    chunk = 256
    assert b_per_w % (2 * chunk) == 0
    out = _lookup(x_flat, embedding_table, b_per_w, chunk)
    return out.reshape(*orig_shape, 2 * EMBED)[:, :, :EMBED]


# speedup vs baseline: 1.1607x; 1.1607x over previous
import functools

import jax
import jax.numpy as jnp
from jax import lax
from jax.experimental import pallas as pl
from jax.experimental.pallas import tpu as pltpu
from jax.experimental.pallas import tpu_sc as plsc

EMBED = 64
SCALE = 8.0

_info = plsc.get_sparse_core_info()
_NC, _NS, _L = _info.num_cores, _info.num_subcores, _info.num_lanes
_NW = _NC * _NS


@functools.partial(jax.jit, static_argnames=("b_per_w", "chunk"))
def _lookup(x_flat, table, b_per_w, chunk):
    n_chunks = b_per_w // chunk
    mesh = plsc.VectorSubcoreMesh(core_axis_name="c", subcore_axis_name="s")

    @functools.partial(
        pl.kernel,
        out_type=jax.ShapeDtypeStruct((x_flat.shape[0], EMBED), jnp.float32),
        mesh=mesh,
        scratch_types=[
            pltpu.VMEM((chunk,), jnp.int32),
            pltpu.VMEM((chunk,), jnp.int32),
            pltpu.VMEM((chunk, EMBED), jnp.float32),
            pltpu.VMEM((chunk, EMBED), jnp.float32),
            pltpu.SemaphoreType.DMA,
            pltpu.SemaphoreType.DMA,
        ],
        compiler_params=pltpu.CompilerParams(use_tc_tiling_on_sc=False),
    )
    def k(x_hbm, table_hbm, out_hbm, idx0, idx1, rows0, rows1, sem0, sem1):
        wid = lax.axis_index("s") * _NC + lax.axis_index("c")
        base = wid * b_per_w
        idx_v = (idx0, idx1)
        rows_v = (rows0, rows1)
        sems = (sem0, sem1)

        def start_gather(g, b):
            off = base + g * chunk
            pltpu.sync_copy(x_hbm.at[pl.ds(off, chunk)], idx_v[b])
            return pltpu.async_copy(table_hbm.at[idx_v[b]], rows_v[b], sems[b])

        def scale_and_store(g, b):
            rv = rows_v[b]

            def scale_rows(r2, c2):
                for u in range(4):
                    for c in range(EMBED // _L):
                        sl = pl.ds(c * _L, _L)
                        rv[4 * r2 + u, sl] = rv[4 * r2 + u, sl] * SCALE
                return c2

            lax.fori_loop(0, chunk // 4, scale_rows, 0)
            pltpu.sync_copy(rv, out_hbm.at[pl.ds(base + g * chunk, chunk)])

        start_gather(0, 0)

        def pair_body(t, carry):
            for b in range(2):
                g = 2 * t + b
                pltpu.make_async_copy(
                    table_hbm.at[idx_v[b]], rows_v[b], sems[b]
                ).wait()

                @pl.when(g + 1 < n_chunks)
                def _prefetch():
                    start_gather(g + 1, 1 - b)

                scale_and_store(g, b)
            return carry

        lax.fori_loop(0, n_chunks // 2, pair_body, 0)

    return k(x_flat, table)


def kernel(x, embedding_table):
    orig_shape = x.shape
    x_flat = x.reshape(-1).astype(jnp.int32)
    b = x_flat.shape[0]
    b_per_w = b // _NW
    chunk = 800
    assert b_per_w % (2 * chunk) == 0
    out = _lookup(x_flat, embedding_table, b_per_w, chunk)
    return out.reshape(*orig_shape, EMBED)
